# baseline (speedup 1.0000x reference)
import functools

import jax
import jax.numpy as jnp
from jax import lax
from jax.experimental import pallas as pl
from jax.experimental.pallas import tpu as pltpu
from jax.experimental.pallas import tpu_sc as plsc

B, H, W, DIN = 2, 64, 64, 256
HID = 1024
DBINS = 64
C = 32
MAXK = 16384
RES = 256
HWP = H * W
NPIX = B * HWP
NVOX = HWP * DBINS
K = MAXK
NOUT = B * K

_DENSE_R = 128
_PREC = None


def _lift_body(x_ref, w1_ref, b1_ref, wd_ref, bd_ref, wf_ref, bf_ref,
               dlog_ref, f2d_ref):
    x = x_ref[...]
    h = jax.nn.gelu(jnp.dot(x, w1_ref[...], preferred_element_type=jnp.float32)
                    + b1_ref[...])
    dlog_ref[...] = jnp.dot(h, wd_ref[...], preferred_element_type=jnp.float32) + bd_ref[...]
    f2d_ref[...] = jnp.dot(h, wf_ref[...], preferred_element_type=jnp.float32) + bf_ref[...]


def _lift(feats2, lift_W1, lift_b1, depth_W, depth_b, feat_W, feat_b,
          interpret=False):
    R = _DENSE_R
    grid = (NPIX // R,)
    full = lambda shape: pl.BlockSpec(shape, lambda i: (0,) * len(shape))
    return pl.pallas_call(
        _lift_body,
        grid=grid,
        in_specs=[
            pl.BlockSpec((R, DIN), lambda i: (i, 0)),
            full((DIN, HID)), full((1, HID)),
            full((HID, DBINS)), full((1, DBINS)),
            full((HID, C)), full((1, C)),
        ],
        out_specs=[
            pl.BlockSpec((R, DBINS), lambda i: (i, 0)),
            pl.BlockSpec((R, C), lambda i: (i, 0)),
        ],
        out_shape=[
            jax.ShapeDtypeStruct((NPIX, DBINS), jnp.float32),
            jax.ShapeDtypeStruct((NPIX, C), jnp.float32),
        ],
        interpret=interpret,
    )(feats2, lift_W1, lift_b1[None, :], depth_W, depth_b[None, :],
      feat_W, feat_b[None, :])


def _conf_body(dp_ref, f2d_ref, cw1_ref, cb1_ref, cw2_ref, cb2_ref, dmask_ref,
               conf_ref):
    dp = dp_ref[...]
    f2d = f2d_ref[...]
    t1 = jnp.concatenate([dp] * DBINS, axis=0)
    t2 = t1 * dmask_ref[...]
    dpw = jnp.dot(t2, jnp.ones((DBINS, C), jnp.float32),
                  preferred_element_type=jnp.float32,
                  precision=jax.lax.Precision.HIGHEST)
    vf = dpw * jnp.concatenate([f2d] * DBINS, axis=0)
    ch = jax.nn.gelu(jnp.dot(vf, cw1_ref[...], preferred_element_type=jnp.float32)
                     + cb1_ref[...])
    s = jnp.dot(ch, cw2_ref[...], preferred_element_type=jnp.float32) + cb2_ref[...]
    conf_ref[...] = jax.nn.sigmoid(s) * dpw[:, 0:1]


def _conf_forward(dp, f2d, conf_W1, conf_b1, conf_W2, conf_b2, interpret=False):
    R = _DENSE_R
    nblk = NPIX // R
    grid = (nblk,)
    dmask = jnp.repeat(jnp.eye(DBINS, dtype=jnp.float32), R, axis=0)
    full = lambda shape: pl.BlockSpec(shape, lambda i: (0,) * len(shape))
    conf_s = pl.pallas_call(
        _conf_body,
        grid=grid,
        in_specs=[
            pl.BlockSpec((R, DBINS), lambda i: (i, 0)),
            pl.BlockSpec((R, C), lambda i: (i, 0)),
            full((C, 64)), full((1, 64)),
            full((64, 1)), full((1, 1)),
            full((DBINS * R, DBINS)),
        ],
        out_specs=pl.BlockSpec((DBINS * R, 1), lambda i: (i, 0)),
        out_shape=jax.ShapeDtypeStruct((NPIX * DBINS, 1), jnp.float32),
        interpret=interpret,
    )(dp, f2d, conf_W1, conf_b1[None, :], conf_W2, conf_b2.reshape(1, 1),
      dmask)
    return conf_s.reshape(nblk, DBINS, R).transpose(0, 2, 1).reshape(B, NVOX)


_SC_INFO = None


def _sc_workers():
    global _SC_INFO
    if _SC_INFO is None:
        info = plsc.get_sparse_core_info()
        _SC_INFO = (info.num_cores, info.num_subcores)
    return _SC_INFO


def _sc_gather_body(idx_hbm, comb_hbm, rows_hbm,
                    idx_v, pix_a, pix_b, rows_v, sem, chunk):
    nc, _ = _sc_workers()
    wid = lax.axis_index("s") * nc + lax.axis_index("c")
    base = wid * chunk
    b = base // K
    half = chunk // 2
    pltpu.sync_copy(idx_hbm.at[pl.ds(base, chunk)], idx_v)
    for i in range(chunk // 16):
        v = idx_v[pl.ds(i * 16, 16)]
        pix = (v >> 6) + b * HWP
        if i < half // 16:
            pix_a[pl.ds(i * 16, 16)] = pix
        else:
            pix_b[pl.ds(i * 16 - half, 16)] = pix
    pltpu.async_copy(comb_hbm.at[pix_a], rows_v, sem).wait()
    pltpu.sync_copy(rows_v, rows_hbm.at[pl.ds(base, half)])
    pltpu.async_copy(comb_hbm.at[pix_b], rows_v, sem).wait()
    pltpu.sync_copy(rows_v, rows_hbm.at[pl.ds(base + half, half)])


def _sc_gather(idx_flat, comb):
    nc, ns = _sc_workers()
    nw = nc * ns
    chunk = NOUT // nw
    mesh = plsc.VectorSubcoreMesh(core_axis_name="c", subcore_axis_name="s")
    kern = pl.kernel(
        functools.partial(_sc_gather_body, chunk=chunk),
        mesh=mesh,
        out_type=jax.ShapeDtypeStruct((NOUT, 128), jnp.float32),
        scratch_types=[
            pltpu.VMEM((chunk,), jnp.int32),
            pltpu.VMEM((chunk // 2,), jnp.int32),
            pltpu.VMEM((chunk // 2,), jnp.int32),
            pltpu.VMEM((chunk // 2, 128), jnp.float32),
            pltpu.SemaphoreType.DMA,
        ],
    )
    return kern(idx_flat, comb)


_HEADS_R = 2048


def _heads_body(rows_ref, idx_ref,
                ow1_ref, ob1_ref, ow2_ref, ob2_ref,
                pw1_ref, pb1_ref, pw2_ref, pb2_ref,
                coords_ref, fout_ref, otype_ref, ologits_ref, tf_ref):
    i = pl.program_id(0)
    idx = idx_ref[...]
    d = idx % DBINS
    blk = rows_ref[...]
    dprow = blk[:, :DBINS]
    f2row = blk[:, DBINS:DBINS + C]
    oh = (lax.broadcasted_iota(jnp.int32, (_HEADS_R, DBINS), 1) == d
          ).astype(jnp.float32)
    dsel = jnp.sum(dprow * oh, axis=1, keepdims=True)
    tf = f2row * dsel
    ol1 = jax.nn.gelu(jnp.dot(tf, ow1_ref[...], preferred_element_type=jnp.float32, precision=_PREC)
                      + ob1_ref[...])
    op_logits = jnp.dot(ol1, ow2_ref[...], preferred_element_type=jnp.float32, precision=_PREC) + ob2_ref[...]
    op_type = jnp.argmax(op_logits, axis=-1).astype(jnp.int32)
    pb = jax.nn.gelu(jnp.dot(tf, pw1_ref[...], preferred_element_type=jnp.float32, precision=_PREC)
                     + pb1_ref[...])
    pbr = jnp.dot(pb, pw2_ref[...], preferred_element_type=jnp.float32, precision=_PREC) + pb2_ref[...]
    fout = jnp.concatenate(
        [jax.nn.sigmoid(pbr[:, :6]), jnp.tanh(pbr[:, 6:7])], axis=1)
    row = i * _HEADS_R + lax.broadcasted_iota(jnp.int32, (_HEADS_R, 1), 0)
    bidx = row // K
    hw = idx // DBINS
    hh = hw // W
    ww = hw % W
    coords = jnp.concatenate(
        [bidx, hh * (RES // H), ww * (RES // W), d * (RES // DBINS)], axis=1)
    coords_ref[...] = coords
    fout_ref[...] = fout
    otype_ref[...] = op_type[:, None]
    ologits_ref[...] = op_logits
    tf_ref[...] = tf


def _heads(rows, idx_flat, op_W1, op_b1, op_W2, op_b2,
           pbr_W1, pbr_b1, pbr_W2, pbr_b2, interpret=False):
    R = _HEADS_R
    grid = (NOUT // R,)
    full = lambda shape: pl.BlockSpec(shape, lambda i: (0,) * len(shape))
    blk = lambda n: pl.BlockSpec((R, n), lambda i: (i, 0))
    return pl.pallas_call(
        _heads_body,
        grid=grid,
        in_specs=[
            blk(128), blk(1),
            full((C, 32)), full((1, 32)), full((32, 3)), full((1, 3)),
            full((C, 64)), full((1, 64)), full((64, 7)), full((1, 7)),
        ],
        out_specs=[blk(4), blk(7), blk(1), blk(3), blk(C)],
        out_shape=[
            jax.ShapeDtypeStruct((NOUT, 4), jnp.int32),
            jax.ShapeDtypeStruct((NOUT, 7), jnp.float32),
            jax.ShapeDtypeStruct((NOUT, 1), jnp.int32),
            jax.ShapeDtypeStruct((NOUT, 3), jnp.float32),
            jax.ShapeDtypeStruct((NOUT, C), jnp.float32),
        ],
        interpret=interpret,
    )(rows, idx_flat,
      op_W1, op_b1[None, :], op_W2, op_b2[None, :],
      pbr_W1, pbr_b1[None, :], pbr_W2, pbr_b2[None, :])


def kernel(features, lift_W1, lift_b1, depth_W, depth_b, feat_W, feat_b,
           conf_W1, conf_b1, conf_W2, conf_b2, op_W1, op_b1, op_W2, op_b2,
           pbr_W1, pbr_b1, pbr_W2, pbr_b2):
    h = jax.nn.gelu(features @ lift_W1 + lift_b1)
    dp4 = jax.nn.softmax(h @ depth_W + depth_b, axis=-1)
    f2d4 = h @ feat_W + feat_b
    dp4, f2d4 = lax.optimization_barrier((dp4, f2d4))
    dp = dp4.reshape(NPIX, DBINS)
    f2d = f2d4.reshape(NPIX, C)
    comb = jnp.concatenate(
        [dp, f2d, jnp.zeros((NPIX, 128 - DBINS - C), jnp.float32)], axis=1)
    conf = _conf_forward(dp, f2d, conf_W1, conf_b1, conf_W2, conf_b2)
    topk_conf, topk_idx = lax.top_k(conf, K)
    idx_flat = topk_idx.reshape(-1)
    rows = _sc_gather(idx_flat, comb)
    coords, fout, otype, ologits, tf = _heads(
        rows, idx_flat[:, None],
        op_W1, op_b1, op_W2, op_b2, pbr_W1, pbr_b1, pbr_W2, pbr_b2)
    return (coords, fout, otype.reshape(-1), topk_conf.reshape(-1),
            ologits, tf)


# speedup vs baseline: 2.1324x; 2.1324x over previous
import functools

import jax
import jax.numpy as jnp
from jax import lax
from jax.experimental import pallas as pl
from jax.experimental.pallas import tpu as pltpu
from jax.experimental.pallas import tpu_sc as plsc

B, H, W, DIN = 2, 64, 64, 256
HID = 1024
DBINS = 64
C = 32
MAXK = 16384
RES = 256
HWP = H * W
NPIX = B * HWP
NVOX = HWP * DBINS
K = MAXK
NOUT = B * K

_DENSE_R = 128
_PREC = None


def _lift_body(x_ref, w1_ref, b1_ref, wd_ref, bd_ref, wf_ref, bf_ref,
               dlog_ref, f2d_ref):
    x = x_ref[...]
    h = jax.nn.gelu(jnp.dot(x, w1_ref[...], preferred_element_type=jnp.float32)
                    + b1_ref[...])
    dlog_ref[...] = jnp.dot(h, wd_ref[...], preferred_element_type=jnp.float32) + bd_ref[...]
    f2d_ref[...] = jnp.dot(h, wf_ref[...], preferred_element_type=jnp.float32) + bf_ref[...]


def _lift(feats2, lift_W1, lift_b1, depth_W, depth_b, feat_W, feat_b,
          interpret=False):
    R = _DENSE_R
    grid = (NPIX // R,)
    full = lambda shape: pl.BlockSpec(shape, lambda i: (0,) * len(shape))
    return pl.pallas_call(
        _lift_body,
        grid=grid,
        in_specs=[
            pl.BlockSpec((R, DIN), lambda i: (i, 0)),
            full((DIN, HID)), full((1, HID)),
            full((HID, DBINS)), full((1, DBINS)),
            full((HID, C)), full((1, C)),
        ],
        out_specs=[
            pl.BlockSpec((R, DBINS), lambda i: (i, 0)),
            pl.BlockSpec((R, C), lambda i: (i, 0)),
        ],
        out_shape=[
            jax.ShapeDtypeStruct((NPIX, DBINS), jnp.float32),
            jax.ShapeDtypeStruct((NPIX, C), jnp.float32),
        ],
        interpret=interpret,
    )(feats2, lift_W1, lift_b1[None, :], depth_W, depth_b[None, :],
      feat_W, feat_b[None, :])


def _conf_body(dp_ref, f2d_ref, cw1_ref, cb1_ref, cw2_ref, cb2_ref, dmask_ref,
               conf_ref):
    dp = dp_ref[...]
    f2d = f2d_ref[...]
    t1 = jnp.concatenate([dp] * DBINS, axis=0)
    t2 = t1 * dmask_ref[...]
    dpw = jnp.dot(t2, jnp.ones((DBINS, C), jnp.float32),
                  preferred_element_type=jnp.float32,
                  precision=jax.lax.Precision.HIGHEST)
    vf = dpw * jnp.concatenate([f2d] * DBINS, axis=0)
    ch = jax.nn.gelu(jnp.dot(vf, cw1_ref[...], preferred_element_type=jnp.float32)
                     + cb1_ref[...])
    s = jnp.dot(ch, cw2_ref[...], preferred_element_type=jnp.float32) + cb2_ref[...]
    conf_ref[...] = jax.nn.sigmoid(s) * dpw[:, 0:1]


def _conf_forward(dp, f2d, conf_W1, conf_b1, conf_W2, conf_b2, interpret=False):
    R = _DENSE_R
    nblk = NPIX // R
    grid = (nblk,)
    dmask = jnp.repeat(jnp.eye(DBINS, dtype=jnp.float32), R, axis=0)
    full = lambda shape: pl.BlockSpec(shape, lambda i: (0,) * len(shape))
    conf_s = pl.pallas_call(
        _conf_body,
        grid=grid,
        in_specs=[
            pl.BlockSpec((R, DBINS), lambda i: (i, 0)),
            pl.BlockSpec((R, C), lambda i: (i, 0)),
            full((C, 64)), full((1, 64)),
            full((64, 1)), full((1, 1)),
            full((DBINS * R, DBINS)),
        ],
        out_specs=pl.BlockSpec((DBINS * R, 1), lambda i: (i, 0)),
        out_shape=jax.ShapeDtypeStruct((NPIX * DBINS, 1), jnp.float32),
        interpret=interpret,
    )(dp, f2d, conf_W1, conf_b1[None, :], conf_W2, conf_b2.reshape(1, 1),
      dmask)
    return conf_s.reshape(nblk, DBINS, R).transpose(0, 2, 1).reshape(B, NVOX)


_SC_INFO = None


def _sc_workers():
    global _SC_INFO
    if _SC_INFO is None:
        info = plsc.get_sparse_core_info()
        _SC_INFO = (info.num_cores, info.num_subcores)
    return _SC_INFO


def _sc_gather_body(idx_hbm, comb_hbm, rows_hbm,
                    idx_v, pix_a, pix_b, rows_v, sem, chunk):
    nc, _ = _sc_workers()
    wid = lax.axis_index("s") * nc + lax.axis_index("c")
    base = wid * chunk
    b = base // K
    half = chunk // 2
    pltpu.sync_copy(idx_hbm.at[pl.ds(base, chunk)], idx_v)
    for i in range(chunk // 16):
        v = idx_v[pl.ds(i * 16, 16)]
        pix = (v >> 6) + b * HWP
        if i < half // 16:
            pix_a[pl.ds(i * 16, 16)] = pix
        else:
            pix_b[pl.ds(i * 16 - half, 16)] = pix
    pltpu.async_copy(comb_hbm.at[pix_a], rows_v, sem).wait()
    pltpu.sync_copy(rows_v, rows_hbm.at[pl.ds(base, half)])
    pltpu.async_copy(comb_hbm.at[pix_b], rows_v, sem).wait()
    pltpu.sync_copy(rows_v, rows_hbm.at[pl.ds(base + half, half)])


def _sc_gather(idx_flat, comb):
    nc, ns = _sc_workers()
    nw = nc * ns
    chunk = NOUT // nw
    mesh = plsc.VectorSubcoreMesh(core_axis_name="c", subcore_axis_name="s")
    kern = pl.kernel(
        functools.partial(_sc_gather_body, chunk=chunk),
        mesh=mesh,
        out_type=jax.ShapeDtypeStruct((NOUT, 128), jnp.float32),
        scratch_types=[
            pltpu.VMEM((chunk,), jnp.int32),
            pltpu.VMEM((chunk // 2,), jnp.int32),
            pltpu.VMEM((chunk // 2,), jnp.int32),
            pltpu.VMEM((chunk // 2, 128), jnp.float32),
            pltpu.SemaphoreType.DMA,
        ],
    )
    return kern(idx_flat, comb)


_HEADS_R = 2048


def _heads_body(rows_ref, idx_ref,
                ow1_ref, ob1_ref, ow2_ref, ob2_ref,
                pw1_ref, pb1_ref, pw2_ref, pb2_ref,
                coords_ref, fout_ref, otype_ref, ologits_ref, tf_ref):
    i = pl.program_id(0)
    idx = idx_ref[...]
    d = idx % DBINS
    blk = rows_ref[...]
    dprow = blk[:, :DBINS]
    f2row = blk[:, DBINS:DBINS + C]
    oh = (lax.broadcasted_iota(jnp.int32, (_HEADS_R, DBINS), 1) == d
          ).astype(jnp.float32)
    dsel = jnp.sum(dprow * oh, axis=1, keepdims=True)
    tf = f2row * dsel
    ol1 = jax.nn.gelu(jnp.dot(tf, ow1_ref[...], preferred_element_type=jnp.float32, precision=_PREC)
                      + ob1_ref[...])
    op_logits = jnp.dot(ol1, ow2_ref[...], preferred_element_type=jnp.float32, precision=_PREC) + ob2_ref[...]
    op_type = jnp.argmax(op_logits, axis=-1).astype(jnp.int32)
    pb = jax.nn.gelu(jnp.dot(tf, pw1_ref[...], preferred_element_type=jnp.float32, precision=_PREC)
                     + pb1_ref[...])
    pbr = jnp.dot(pb, pw2_ref[...], preferred_element_type=jnp.float32, precision=_PREC) + pb2_ref[...]
    fout = jnp.concatenate(
        [jax.nn.sigmoid(pbr[:, :6]), jnp.tanh(pbr[:, 6:7])], axis=1)
    row = i * _HEADS_R + lax.broadcasted_iota(jnp.int32, (_HEADS_R, 1), 0)
    bidx = row // K
    hw = idx // DBINS
    hh = hw // W
    ww = hw % W
    coords = jnp.concatenate(
        [bidx, hh * (RES // H), ww * (RES // W), d * (RES // DBINS)], axis=1)
    coords_ref[...] = coords
    fout_ref[...] = fout
    otype_ref[...] = op_type[:, None]
    ologits_ref[...] = op_logits
    tf_ref[...] = tf


def _heads(rows, idx_flat, op_W1, op_b1, op_W2, op_b2,
           pbr_W1, pbr_b1, pbr_W2, pbr_b2, interpret=False):
    R = _HEADS_R
    grid = (NOUT // R,)
    full = lambda shape: pl.BlockSpec(shape, lambda i: (0,) * len(shape))
    blk = lambda n: pl.BlockSpec((R, n), lambda i: (i, 0))
    return pl.pallas_call(
        _heads_body,
        grid=grid,
        in_specs=[
            blk(128), blk(1),
            full((C, 32)), full((1, 32)), full((32, 3)), full((1, 3)),
            full((C, 64)), full((1, 64)), full((64, 7)), full((1, 7)),
        ],
        out_specs=[blk(4), blk(7), blk(1), blk(3), blk(C)],
        out_shape=[
            jax.ShapeDtypeStruct((NOUT, 4), jnp.int32),
            jax.ShapeDtypeStruct((NOUT, 7), jnp.float32),
            jax.ShapeDtypeStruct((NOUT, 1), jnp.int32),
            jax.ShapeDtypeStruct((NOUT, 3), jnp.float32),
            jax.ShapeDtypeStruct((NOUT, C), jnp.float32),
        ],
        interpret=interpret,
    )(rows, idx_flat,
      op_W1, op_b1[None, :], op_W2, op_b2[None, :],
      pbr_W1, pbr_b1[None, :], pbr_W2, pbr_b2[None, :])


def kernel(features, lift_W1, lift_b1, depth_W, depth_b, feat_W, feat_b,
           conf_W1, conf_b1, conf_W2, conf_b2, op_W1, op_b1, op_W2, op_b2,
           pbr_W1, pbr_b1, pbr_W2, pbr_b2):
    h = jax.nn.gelu(features @ lift_W1 + lift_b1)
    dp4 = jax.nn.softmax(h @ depth_W + depth_b, axis=-1)
    f2d4 = h @ feat_W + feat_b
    dp4, f2d4 = lax.optimization_barrier((dp4, f2d4))
    dp = dp4.reshape(NPIX, DBINS)
    f2d = f2d4.reshape(NPIX, C)
    comb = jnp.concatenate(
        [dp, f2d, jnp.zeros((NPIX, 128 - DBINS - C), jnp.float32)], axis=1)
    conf = _conf_forward(dp, f2d, conf_W1, conf_b1, conf_W2, conf_b2)
    M = 16
    conf4 = conf.reshape(B, HWP, DBINS)
    cv, ci = lax.top_k(conf4, M + 1)
    cand = cv[:, :, :M].reshape(B, HWP * M)
    cd = ci[:, :, :M].reshape(B, HWP * M)
    tkc, tkj = lax.top_k(cand, K)
    pixk = tkj // M
    dk = jnp.take_along_axis(cd, tkj, axis=1)
    idx_fast = pixk * DBINS + dk
    excl_bound = jnp.max(cv[:, :, M], axis=1)
    valid = jnp.all(excl_bound < tkc[:, K - 1])
    topk_conf, topk_idx = lax.cond(
        valid,
        lambda: (tkc, idx_fast),
        lambda: tuple(lax.top_k(conf, K)))
    idx_flat = topk_idx.reshape(-1)
    rows = _sc_gather(idx_flat, comb)
    coords, fout, otype, ologits, tf = _heads(
        rows, idx_flat[:, None],
        op_W1, op_b1, op_W2, op_b2, pbr_W1, pbr_b1, pbr_W2, pbr_b2)
    return (coords, fout, otype.reshape(-1), topk_conf.reshape(-1),
            ologits, tf)


# speedup vs baseline: 2.4645x; 1.1557x over previous
import functools

import jax
import jax.numpy as jnp
from jax import lax
from jax.experimental import pallas as pl
from jax.experimental.pallas import tpu as pltpu
from jax.experimental.pallas import tpu_sc as plsc

B, H, W, DIN = 2, 64, 64, 256
HID = 1024
DBINS = 64
C = 32
MAXK = 16384
RES = 256
HWP = H * W
NPIX = B * HWP
NVOX = HWP * DBINS
K = MAXK
NOUT = B * K

_DENSE_R = 128
_PREC = None


def _lift_body(x_ref, w1_ref, b1_ref, wd_ref, bd_ref, wf_ref, bf_ref,
               dlog_ref, f2d_ref):
    x = x_ref[...]
    h = jax.nn.gelu(jnp.dot(x, w1_ref[...], preferred_element_type=jnp.float32)
                    + b1_ref[...])
    dlog_ref[...] = jnp.dot(h, wd_ref[...], preferred_element_type=jnp.float32) + bd_ref[...]
    f2d_ref[...] = jnp.dot(h, wf_ref[...], preferred_element_type=jnp.float32) + bf_ref[...]


def _lift(feats2, lift_W1, lift_b1, depth_W, depth_b, feat_W, feat_b,
          interpret=False):
    R = _DENSE_R
    grid = (NPIX // R,)
    full = lambda shape: pl.BlockSpec(shape, lambda i: (0,) * len(shape))
    return pl.pallas_call(
        _lift_body,
        grid=grid,
        in_specs=[
            pl.BlockSpec((R, DIN), lambda i: (i, 0)),
            full((DIN, HID)), full((1, HID)),
            full((HID, DBINS)), full((1, DBINS)),
            full((HID, C)), full((1, C)),
        ],
        out_specs=[
            pl.BlockSpec((R, DBINS), lambda i: (i, 0)),
            pl.BlockSpec((R, C), lambda i: (i, 0)),
        ],
        out_shape=[
            jax.ShapeDtypeStruct((NPIX, DBINS), jnp.float32),
            jax.ShapeDtypeStruct((NPIX, C), jnp.float32),
        ],
        interpret=interpret,
    )(feats2, lift_W1, lift_b1[None, :], depth_W, depth_b[None, :],
      feat_W, feat_b[None, :])


def _conf_body(dp_ref, f2d_ref, cw1_ref, cb1_ref, cw2_ref, cb2_ref,
               conf_ref):
    dp = dp_ref[...]
    f2d = f2d_ref[...]
    dcol = jnp.concatenate([dp[:, d:d + 1] for d in range(DBINS)], axis=0)
    vf = jnp.concatenate([dp[:, d:d + 1] * f2d for d in range(DBINS)], axis=0)
    ch = jax.nn.gelu(jnp.dot(vf, cw1_ref[...], preferred_element_type=jnp.float32)
                     + cb1_ref[...])
    s = jnp.dot(ch, cw2_ref[...], preferred_element_type=jnp.float32) + cb2_ref[...]
    conf_ref[...] = jax.nn.sigmoid(s) * dcol


def _conf_forward(dp, f2d, conf_W1, conf_b1, conf_W2, conf_b2, interpret=False):
    R = _DENSE_R
    nblk = NPIX // R
    grid = (nblk,)
    full = lambda shape: pl.BlockSpec(shape, lambda i: (0,) * len(shape))
    conf_s = pl.pallas_call(
        _conf_body,
        grid=grid,
        in_specs=[
            pl.BlockSpec((R, DBINS), lambda i: (i, 0)),
            pl.BlockSpec((R, C), lambda i: (i, 0)),
            full((C, 64)), full((1, 64)),
            full((64, 1)), full((1, 1)),
        ],
        out_specs=pl.BlockSpec((DBINS * R, 1), lambda i: (i, 0)),
        out_shape=jax.ShapeDtypeStruct((NPIX * DBINS, 1), jnp.float32),
        interpret=interpret,
    )(dp, f2d, conf_W1, conf_b1[None, :], conf_W2, conf_b2.reshape(1, 1))
    return conf_s.reshape(nblk, DBINS, R).transpose(0, 2, 1).reshape(B, NVOX)


_SC_INFO = None


def _sc_workers():
    global _SC_INFO
    if _SC_INFO is None:
        info = plsc.get_sparse_core_info()
        _SC_INFO = (info.num_cores, info.num_subcores)
    return _SC_INFO


def _sc_gather_body(idx_hbm, comb_hbm, rows_hbm,
                    idx_v, pix_a, pix_b, rows_v, sem, chunk):
    nc, _ = _sc_workers()
    wid = lax.axis_index("s") * nc + lax.axis_index("c")
    base = wid * chunk
    b = base // K
    half = chunk // 2
    pltpu.sync_copy(idx_hbm.at[pl.ds(base, chunk)], idx_v)
    for i in range(chunk // 16):
        v = idx_v[pl.ds(i * 16, 16)]
        pix = (v >> 6) + b * HWP
        if i < half // 16:
            pix_a[pl.ds(i * 16, 16)] = pix
        else:
            pix_b[pl.ds(i * 16 - half, 16)] = pix
    pltpu.async_copy(comb_hbm.at[pix_a], rows_v, sem).wait()
    pltpu.sync_copy(rows_v, rows_hbm.at[pl.ds(base, half)])
    pltpu.async_copy(comb_hbm.at[pix_b], rows_v, sem).wait()
    pltpu.sync_copy(rows_v, rows_hbm.at[pl.ds(base + half, half)])


def _sc_gather(idx_flat, comb):
    nc, ns = _sc_workers()
    nw = nc * ns
    chunk = NOUT // nw
    mesh = plsc.VectorSubcoreMesh(core_axis_name="c", subcore_axis_name="s")
    kern = pl.kernel(
        functools.partial(_sc_gather_body, chunk=chunk),
        mesh=mesh,
        out_type=jax.ShapeDtypeStruct((NOUT, 128), jnp.float32),
        scratch_types=[
            pltpu.VMEM((chunk,), jnp.int32),
            pltpu.VMEM((chunk // 2,), jnp.int32),
            pltpu.VMEM((chunk // 2,), jnp.int32),
            pltpu.VMEM((chunk // 2, 128), jnp.float32),
            pltpu.SemaphoreType.DMA,
        ],
    )
    return kern(idx_flat, comb)


_HEADS_R = 2048


def _heads_body(rows_ref, idx_ref,
                ow1_ref, ob1_ref, ow2_ref, ob2_ref,
                pw1_ref, pb1_ref, pw2_ref, pb2_ref,
                coords_ref, fout_ref, otype_ref, ologits_ref, tf_ref):
    i = pl.program_id(0)
    idx = idx_ref[...]
    d = idx % DBINS
    blk = rows_ref[...]
    dprow = blk[:, :DBINS]
    f2row = blk[:, DBINS:DBINS + C]
    oh = (lax.broadcasted_iota(jnp.int32, (_HEADS_R, DBINS), 1) == d
          ).astype(jnp.float32)
    dsel = jnp.sum(dprow * oh, axis=1, keepdims=True)
    tf = f2row * dsel
    ol1 = jax.nn.gelu(jnp.dot(tf, ow1_ref[...], preferred_element_type=jnp.float32, precision=_PREC)
                      + ob1_ref[...])
    op_logits = jnp.dot(ol1, ow2_ref[...], preferred_element_type=jnp.float32, precision=_PREC) + ob2_ref[...]
    op_type = jnp.argmax(op_logits, axis=-1).astype(jnp.int32)
    pb = jax.nn.gelu(jnp.dot(tf, pw1_ref[...], preferred_element_type=jnp.float32, precision=_PREC)
                     + pb1_ref[...])
    pbr = jnp.dot(pb, pw2_ref[...], preferred_element_type=jnp.float32, precision=_PREC) + pb2_ref[...]
    fout = jnp.concatenate(
        [jax.nn.sigmoid(pbr[:, :6]), jnp.tanh(pbr[:, 6:7])], axis=1)
    row = i * _HEADS_R + lax.broadcasted_iota(jnp.int32, (_HEADS_R, 1), 0)
    bidx = row // K
    hw = idx // DBINS
    hh = hw // W
    ww = hw % W
    coords = jnp.concatenate(
        [bidx, hh * (RES // H), ww * (RES // W), d * (RES // DBINS)], axis=1)
    coords_ref[...] = coords
    fout_ref[...] = fout
    otype_ref[...] = op_type[:, None]
    ologits_ref[...] = op_logits
    tf_ref[...] = tf


def _heads(rows, idx_flat, op_W1, op_b1, op_W2, op_b2,
           pbr_W1, pbr_b1, pbr_W2, pbr_b2, interpret=False):
    R = _HEADS_R
    grid = (NOUT // R,)
    full = lambda shape: pl.BlockSpec(shape, lambda i: (0,) * len(shape))
    blk = lambda n: pl.BlockSpec((R, n), lambda i: (i, 0))
    return pl.pallas_call(
        _heads_body,
        grid=grid,
        in_specs=[
            blk(128), blk(1),
            full((C, 32)), full((1, 32)), full((32, 3)), full((1, 3)),
            full((C, 64)), full((1, 64)), full((64, 7)), full((1, 7)),
        ],
        out_specs=[blk(4), blk(7), blk(1), blk(3), blk(C)],
        out_shape=[
            jax.ShapeDtypeStruct((NOUT, 4), jnp.int32),
            jax.ShapeDtypeStruct((NOUT, 7), jnp.float32),
            jax.ShapeDtypeStruct((NOUT, 1), jnp.int32),
            jax.ShapeDtypeStruct((NOUT, 3), jnp.float32),
            jax.ShapeDtypeStruct((NOUT, C), jnp.float32),
        ],
        interpret=interpret,
    )(rows, idx_flat,
      op_W1, op_b1[None, :], op_W2, op_b2[None, :],
      pbr_W1, pbr_b1[None, :], pbr_W2, pbr_b2[None, :])


def kernel(features, lift_W1, lift_b1, depth_W, depth_b, feat_W, feat_b,
           conf_W1, conf_b1, conf_W2, conf_b2, op_W1, op_b1, op_W2, op_b2,
           pbr_W1, pbr_b1, pbr_W2, pbr_b2):
    h = jax.nn.gelu(features @ lift_W1 + lift_b1)
    dp4 = jax.nn.softmax(h @ depth_W + depth_b, axis=-1)
    f2d4 = h @ feat_W + feat_b
    dp4, f2d4 = lax.optimization_barrier((dp4, f2d4))
    dp = dp4.reshape(NPIX, DBINS)
    f2d = f2d4.reshape(NPIX, C)
    comb = jnp.concatenate(
        [dp, f2d, jnp.zeros((NPIX, 128 - DBINS - C), jnp.float32)], axis=1)
    conf = _conf_forward(dp, f2d, conf_W1, conf_b1, conf_W2, conf_b2)
    M = 16
    conf4 = conf.reshape(B, HWP, DBINS)
    cv, ci = lax.top_k(conf4, M + 1)
    cand = cv[:, :, :M].reshape(B, HWP * M)
    cd = ci[:, :, :M].reshape(B, HWP * M)
    tkc, tkj = lax.top_k(cand, K)
    pixk = tkj // M
    dk = jnp.take_along_axis(cd, tkj, axis=1)
    idx_fast = pixk * DBINS + dk
    excl_bound = jnp.max(cv[:, :, M], axis=1)
    valid = jnp.all(excl_bound < tkc[:, K - 1])
    topk_conf, topk_idx = lax.cond(
        valid,
        lambda: (tkc, idx_fast),
        lambda: tuple(lax.top_k(conf, K)))
    idx_flat = topk_idx.reshape(-1)
    rows = _sc_gather(idx_flat, comb)
    coords, fout, otype, ologits, tf = _heads(
        rows, idx_flat[:, None],
        op_W1, op_b1, op_W2, op_b2, pbr_W1, pbr_b1, pbr_W2, pbr_b2)
    return (coords, fout, otype.reshape(-1), topk_conf.reshape(-1),
            ologits, tf)


# speedup vs baseline: 2.5143x; 1.0202x over previous
import functools

import jax
import jax.numpy as jnp
from jax import lax
from jax.experimental import pallas as pl
from jax.experimental.pallas import tpu as pltpu
from jax.experimental.pallas import tpu_sc as plsc

B, H, W, DIN = 2, 64, 64, 256
HID = 1024
DBINS = 64
C = 32
MAXK = 16384
RES = 256
HWP = H * W
NPIX = B * HWP
NVOX = HWP * DBINS
K = MAXK
NOUT = B * K

_DENSE_R = 256
_PREC = None


def _lift_body(x_ref, w1_ref, b1_ref, wd_ref, bd_ref, wf_ref, bf_ref,
               dlog_ref, f2d_ref):
    x = x_ref[...]
    h = jax.nn.gelu(jnp.dot(x, w1_ref[...], preferred_element_type=jnp.float32)
                    + b1_ref[...])
    dlog_ref[...] = jnp.dot(h, wd_ref[...], preferred_element_type=jnp.float32) + bd_ref[...]
    f2d_ref[...] = jnp.dot(h, wf_ref[...], preferred_element_type=jnp.float32) + bf_ref[...]


def _lift(feats2, lift_W1, lift_b1, depth_W, depth_b, feat_W, feat_b,
          interpret=False):
    R = _DENSE_R
    grid = (NPIX // R,)
    full = lambda shape: pl.BlockSpec(shape, lambda i: (0,) * len(shape))
    return pl.pallas_call(
        _lift_body,
        grid=grid,
        in_specs=[
            pl.BlockSpec((R, DIN), lambda i: (i, 0)),
            full((DIN, HID)), full((1, HID)),
            full((HID, DBINS)), full((1, DBINS)),
            full((HID, C)), full((1, C)),
        ],
        out_specs=[
            pl.BlockSpec((R, DBINS), lambda i: (i, 0)),
            pl.BlockSpec((R, C), lambda i: (i, 0)),
        ],
        out_shape=[
            jax.ShapeDtypeStruct((NPIX, DBINS), jnp.float32),
            jax.ShapeDtypeStruct((NPIX, C), jnp.float32),
        ],
        interpret=interpret,
    )(feats2, lift_W1, lift_b1[None, :], depth_W, depth_b[None, :],
      feat_W, feat_b[None, :])


def _conf_body(dp_ref, f2d_ref, cw1_ref, cb1_ref, cw2_ref, cb2_ref,
               conf_ref):
    dp = dp_ref[...]
    f2d = f2d_ref[...]
    dcol = jnp.concatenate([dp[:, d:d + 1] for d in range(DBINS)], axis=0)
    vf = jnp.concatenate([dp[:, d:d + 1] * f2d for d in range(DBINS)], axis=0)
    ch = jax.nn.gelu(jnp.dot(vf, cw1_ref[...], preferred_element_type=jnp.float32)
                     + cb1_ref[...])
    s = jnp.dot(ch, cw2_ref[...], preferred_element_type=jnp.float32) + cb2_ref[...]
    conf_ref[...] = jax.nn.sigmoid(s) * dcol


def _conf_forward(dp, f2d, conf_W1, conf_b1, conf_W2, conf_b2, interpret=False):
    R = _DENSE_R
    nblk = NPIX // R
    grid = (nblk,)
    full = lambda shape: pl.BlockSpec(shape, lambda i: (0,) * len(shape))
    conf_s = pl.pallas_call(
        _conf_body,
        grid=grid,
        in_specs=[
            pl.BlockSpec((R, DBINS), lambda i: (i, 0)),
            pl.BlockSpec((R, C), lambda i: (i, 0)),
            full((C, 64)), full((1, 64)),
            full((64, 1)), full((1, 1)),
        ],
        out_specs=pl.BlockSpec((DBINS * R, 1), lambda i: (i, 0)),
        out_shape=jax.ShapeDtypeStruct((NPIX * DBINS, 1), jnp.float32),
        interpret=interpret,
    )(dp, f2d, conf_W1, conf_b1[None, :], conf_W2, conf_b2.reshape(1, 1))
    return conf_s.reshape(nblk, DBINS, R).transpose(0, 2, 1).reshape(B, NVOX)


_SC_INFO = None


def _sc_workers():
    global _SC_INFO
    if _SC_INFO is None:
        info = plsc.get_sparse_core_info()
        _SC_INFO = (info.num_cores, info.num_subcores)
    return _SC_INFO


def _sc_gather_body(idx_hbm, comb_hbm, rows_hbm,
                    idx_v, pix_a, pix_b, rows_v, sem, chunk):
    nc, _ = _sc_workers()
    wid = lax.axis_index("s") * nc + lax.axis_index("c")
    base = wid * chunk
    b = base // K
    half = chunk // 2
    pltpu.sync_copy(idx_hbm.at[pl.ds(base, chunk)], idx_v)
    for i in range(chunk // 16):
        v = idx_v[pl.ds(i * 16, 16)]
        pix = (v >> 6) + b * HWP
        if i < half // 16:
            pix_a[pl.ds(i * 16, 16)] = pix
        else:
            pix_b[pl.ds(i * 16 - half, 16)] = pix
    pltpu.async_copy(comb_hbm.at[pix_a], rows_v, sem).wait()
    pltpu.sync_copy(rows_v, rows_hbm.at[pl.ds(base, half)])
    pltpu.async_copy(comb_hbm.at[pix_b], rows_v, sem).wait()
    pltpu.sync_copy(rows_v, rows_hbm.at[pl.ds(base + half, half)])


def _sc_gather(idx_flat, comb):
    nc, ns = _sc_workers()
    nw = nc * ns
    chunk = NOUT // nw
    mesh = plsc.VectorSubcoreMesh(core_axis_name="c", subcore_axis_name="s")
    kern = pl.kernel(
        functools.partial(_sc_gather_body, chunk=chunk),
        mesh=mesh,
        out_type=jax.ShapeDtypeStruct((NOUT, 128), jnp.float32),
        scratch_types=[
            pltpu.VMEM((chunk,), jnp.int32),
            pltpu.VMEM((chunk // 2,), jnp.int32),
            pltpu.VMEM((chunk // 2,), jnp.int32),
            pltpu.VMEM((chunk // 2, 128), jnp.float32),
            pltpu.SemaphoreType.DMA,
        ],
    )
    return kern(idx_flat, comb)


_HEADS_R = 2048


def _heads_body(rows_ref, idx_ref,
                ow1_ref, ob1_ref, ow2_ref, ob2_ref,
                pw1_ref, pb1_ref, pw2_ref, pb2_ref,
                coords_ref, fout_ref, otype_ref, ologits_ref, tf_ref):
    i = pl.program_id(0)
    idx = idx_ref[...]
    d = idx % DBINS
    blk = rows_ref[...]
    dprow = blk[:, :DBINS]
    f2row = blk[:, DBINS:DBINS + C]
    oh = (lax.broadcasted_iota(jnp.int32, (_HEADS_R, DBINS), 1) == d
          ).astype(jnp.float32)
    dsel = jnp.sum(dprow * oh, axis=1, keepdims=True)
    tf = f2row * dsel
    ol1 = jax.nn.gelu(jnp.dot(tf, ow1_ref[...], preferred_element_type=jnp.float32, precision=_PREC)
                      + ob1_ref[...])
    op_logits = jnp.dot(ol1, ow2_ref[...], preferred_element_type=jnp.float32, precision=_PREC) + ob2_ref[...]
    op_type = jnp.argmax(op_logits, axis=-1).astype(jnp.int32)
    pb = jax.nn.gelu(jnp.dot(tf, pw1_ref[...], preferred_element_type=jnp.float32, precision=_PREC)
                     + pb1_ref[...])
    pbr = jnp.dot(pb, pw2_ref[...], preferred_element_type=jnp.float32, precision=_PREC) + pb2_ref[...]
    fout = jnp.concatenate(
        [jax.nn.sigmoid(pbr[:, :6]), jnp.tanh(pbr[:, 6:7])], axis=1)
    row = i * _HEADS_R + lax.broadcasted_iota(jnp.int32, (_HEADS_R, 1), 0)
    bidx = row // K
    hw = idx // DBINS
    hh = hw // W
    ww = hw % W
    coords = jnp.concatenate(
        [bidx, hh * (RES // H), ww * (RES // W), d * (RES // DBINS)], axis=1)
    coords_ref[...] = coords
    fout_ref[...] = fout
    otype_ref[...] = op_type[:, None]
    ologits_ref[...] = op_logits
    tf_ref[...] = tf


def _heads(rows, idx_flat, op_W1, op_b1, op_W2, op_b2,
           pbr_W1, pbr_b1, pbr_W2, pbr_b2, interpret=False):
    R = _HEADS_R
    grid = (NOUT // R,)
    full = lambda shape: pl.BlockSpec(shape, lambda i: (0,) * len(shape))
    blk = lambda n: pl.BlockSpec((R, n), lambda i: (i, 0))
    return pl.pallas_call(
        _heads_body,
        grid=grid,
        in_specs=[
            blk(128), blk(1),
            full((C, 32)), full((1, 32)), full((32, 3)), full((1, 3)),
            full((C, 64)), full((1, 64)), full((64, 7)), full((1, 7)),
        ],
        out_specs=[blk(4), blk(7), blk(1), blk(3), blk(C)],
        out_shape=[
            jax.ShapeDtypeStruct((NOUT, 4), jnp.int32),
            jax.ShapeDtypeStruct((NOUT, 7), jnp.float32),
            jax.ShapeDtypeStruct((NOUT, 1), jnp.int32),
            jax.ShapeDtypeStruct((NOUT, 3), jnp.float32),
            jax.ShapeDtypeStruct((NOUT, C), jnp.float32),
        ],
        interpret=interpret,
    )(rows, idx_flat,
      op_W1, op_b1[None, :], op_W2, op_b2[None, :],
      pbr_W1, pbr_b1[None, :], pbr_W2, pbr_b2[None, :])


def kernel(features, lift_W1, lift_b1, depth_W, depth_b, feat_W, feat_b,
           conf_W1, conf_b1, conf_W2, conf_b2, op_W1, op_b1, op_W2, op_b2,
           pbr_W1, pbr_b1, pbr_W2, pbr_b2):
    h = jax.nn.gelu(features @ lift_W1 + lift_b1)
    dp4 = jax.nn.softmax(h @ depth_W + depth_b, axis=-1)
    f2d4 = h @ feat_W + feat_b
    dp4, f2d4 = lax.optimization_barrier((dp4, f2d4))
    dp = dp4.reshape(NPIX, DBINS)
    f2d = f2d4.reshape(NPIX, C)
    comb = jnp.concatenate(
        [dp, f2d, jnp.zeros((NPIX, 128 - DBINS - C), jnp.float32)], axis=1)
    conf = _conf_forward(dp, f2d, conf_W1, conf_b1, conf_W2, conf_b2)
    M = 16
    conf4 = conf.reshape(B, HWP, DBINS)
    cv, ci = lax.top_k(conf4, M + 1)
    cand = cv[:, :, :M].reshape(B, HWP * M)
    cd = ci[:, :, :M].reshape(B, HWP * M)
    tkc, tkj = lax.top_k(cand, K)
    pixk = tkj // M
    dk = jnp.take_along_axis(cd, tkj, axis=1)
    idx_fast = pixk * DBINS + dk
    excl_bound = jnp.max(cv[:, :, M], axis=1)
    valid = jnp.all(excl_bound < tkc[:, K - 1])
    topk_conf, topk_idx = lax.cond(
        valid,
        lambda: (tkc, idx_fast),
        lambda: tuple(lax.top_k(conf, K)))
    idx_flat = topk_idx.reshape(-1)
    rows = _sc_gather(idx_flat, comb)
    coords, fout, otype, ologits, tf = _heads(
        rows, idx_flat[:, None],
        op_W1, op_b1, op_W2, op_b2, pbr_W1, pbr_b1, pbr_W2, pbr_b2)
    return (coords, fout, otype.reshape(-1), topk_conf.reshape(-1),
            ologits, tf)
